# Initial kernel scaffold; baseline (speedup 1.0000x reference)
import jax
import jax.numpy as jnp
from jax.experimental import pallas as pl


def kernel(feature, A, W1, b1, W2, b2, weight_b, weight_a):
    raise NotImplementedError("write your pallas kernel here")


# speedup vs baseline: 1.3749x; 1.3749x over previous
import jax
import jax.numpy as jnp
from jax.experimental import pallas as pl
from jax.experimental.pallas import tpu as pltpu

N = 10000
F = 128
TM = 200
TM1 = 1000


def _s1_body(f_ref, w1_ref, o_ref):
    fb = f_ref[...].astype(jnp.bfloat16)
    o_ref[...] = jnp.dot(fb, w1_ref[...],
                         preferred_element_type=jnp.float32).astype(jnp.bfloat16)


def _pass1_body(wb_ref, a_ref, s1_ref, b1_ref, w2_ref, u1_ref, s2_ref, fa8_ref):
    wb0 = wb_ref[0, 0]
    wb1 = wb_ref[1, 0]
    m = a_ref[0] * wb0 + a_ref[1] * wb1
    fa8_ref[...] = m.astype(jnp.float8_e4m3fn)
    mb = m.astype(jnp.bfloat16)
    u1 = jnp.dot(mb, s1_ref[...], preferred_element_type=jnp.float32)
    u1 = u1 + b1_ref[...]
    u1_ref[...] = u1
    s2_ref[...] = jnp.dot(u1.astype(jnp.bfloat16), w2_ref[...],
                          preferred_element_type=jnp.float32).astype(jnp.bfloat16)


def _pass2_body(wa_ref, fa8_ref, s2_ref, u1_ref, b2_ref, o_ref):
    wa = wa_ref[0, 0]
    mb = fa8_ref[...].astype(jnp.bfloat16)
    u2 = jnp.dot(mb, s2_ref[...], preferred_element_type=jnp.float32)
    u2 = u2 + b2_ref[...]
    o_ref[...] = (u1_ref[...] + u2 * wa) * 0.5


def kernel(feature, A, W1, b1, W2, b2, weight_b, weight_a):
    w1_bf = W1.astype(jnp.bfloat16)
    w2_bf = W2.astype(jnp.bfloat16)
    b1_2d = b1.reshape(1, F)
    b2_2d = b2.reshape(1, F)

    s1 = pl.pallas_call(
        _s1_body,
        grid=(N // TM1,),
        in_specs=[
            pl.BlockSpec((TM1, F), lambda i: (i, 0)),
            pl.BlockSpec((F, F), lambda i: (0, 0)),
        ],
        out_specs=pl.BlockSpec((TM1, F), lambda i: (i, 0)),
        out_shape=jax.ShapeDtypeStruct((N, F), jnp.bfloat16),
    )(feature, w1_bf)

    u1, s2, fa8 = pl.pallas_call(
        _pass1_body,
        grid=(N // TM,),
        in_specs=[
            pl.BlockSpec(memory_space=pltpu.SMEM),
            pl.BlockSpec((2, TM, N), lambda i: (0, i, 0)),
            pl.BlockSpec((N, F), lambda i: (0, 0)),
            pl.BlockSpec((1, F), lambda i: (0, 0)),
            pl.BlockSpec((F, F), lambda i: (0, 0)),
        ],
        out_specs=[
            pl.BlockSpec((TM, F), lambda i: (i, 0)),
            pl.BlockSpec((TM, F), lambda i: (i, 0)),
            pl.BlockSpec((TM, N), lambda i: (i, 0)),
        ],
        out_shape=[
            jax.ShapeDtypeStruct((N, F), jnp.float32),
            jax.ShapeDtypeStruct((N, F), jnp.bfloat16),
            jax.ShapeDtypeStruct((N, N), jnp.float8_e4m3fn),
        ],
    )(weight_b, A, s1, b1_2d, w2_bf)

    out = pl.pallas_call(
        _pass2_body,
        grid=(N // TM,),
        in_specs=[
            pl.BlockSpec(memory_space=pltpu.SMEM),
            pl.BlockSpec((TM, N), lambda i: (i, 0)),
            pl.BlockSpec((N, F), lambda i: (0, 0)),
            pl.BlockSpec((TM, F), lambda i: (i, 0)),
            pl.BlockSpec((1, F), lambda i: (0, 0)),
        ],
        out_specs=pl.BlockSpec((TM, F), lambda i: (i, 0)),
        out_shape=jax.ShapeDtypeStruct((N, F), jnp.float32),
    )(weight_a, fa8, s2, u1, b2_2d)

    return out


# speedup vs baseline: 1.4028x; 1.0203x over previous
import jax
import jax.numpy as jnp
from jax.experimental import pallas as pl
from jax.experimental.pallas import tpu as pltpu

N = 10000
F = 128
TM = 200
TM1 = 1000


def _s1_body(f_ref, w1_ref, o_ref):
    fb = f_ref[...].astype(jnp.bfloat16)
    o_ref[...] = jnp.dot(fb, w1_ref[...],
                         preferred_element_type=jnp.float32).astype(jnp.bfloat16)


def _pass1_body(wb_ref, a0_ref, a1_ref, s1_ref, b1_ref, w2_ref,
                u1_ref, s2_ref, fa8_ref):
    wb0 = wb_ref[0, 0]
    wb1 = wb_ref[1, 0]
    m = a0_ref[0] * wb0 + a1_ref[0] * wb1
    fa8_ref[...] = m.astype(jnp.float8_e4m3fn)
    mb = m.astype(jnp.bfloat16)
    u1 = jnp.dot(mb, s1_ref[...], preferred_element_type=jnp.float32)
    u1 = u1 + b1_ref[...]
    u1_ref[...] = u1
    s2_ref[...] = jnp.dot(u1.astype(jnp.bfloat16), w2_ref[...],
                          preferred_element_type=jnp.float32).astype(
                              jnp.float8_e4m3fn)


def _pass2_body(wa_ref, fa8_ref, s2_ref, u1_ref, b2_ref, o_ref):
    wa = wa_ref[0, 0]
    u2 = jnp.dot(fa8_ref[...], s2_ref[...],
                 preferred_element_type=jnp.float32)
    u2 = u2 + b2_ref[...]
    o_ref[...] = (u1_ref[...] + u2 * wa) * 0.5


def kernel(feature, A, W1, b1, W2, b2, weight_b, weight_a):
    w1_bf = W1.astype(jnp.bfloat16)
    w2_bf = W2.astype(jnp.bfloat16)
    b1_2d = b1.reshape(1, F)
    b2_2d = b2.reshape(1, F)

    s1 = pl.pallas_call(
        _s1_body,
        grid=(N // TM1,),
        in_specs=[
            pl.BlockSpec((TM1, F), lambda i: (i, 0)),
            pl.BlockSpec((F, F), lambda i: (0, 0)),
        ],
        out_specs=pl.BlockSpec((TM1, F), lambda i: (i, 0)),
        out_shape=jax.ShapeDtypeStruct((N, F), jnp.bfloat16),
    )(feature, w1_bf)

    u1, s2, fa8 = pl.pallas_call(
        _pass1_body,
        grid=(N // TM,),
        in_specs=[
            pl.BlockSpec(memory_space=pltpu.SMEM),
            pl.BlockSpec((1, TM, N), lambda i: (0, i, 0)),
            pl.BlockSpec((1, TM, N), lambda i: (1, i, 0)),
            pl.BlockSpec((N, F), lambda i: (0, 0)),
            pl.BlockSpec((1, F), lambda i: (0, 0)),
            pl.BlockSpec((F, F), lambda i: (0, 0)),
        ],
        out_specs=[
            pl.BlockSpec((TM, F), lambda i: (i, 0)),
            pl.BlockSpec((TM, F), lambda i: (i, 0)),
            pl.BlockSpec((TM, N), lambda i: (i, 0)),
        ],
        out_shape=[
            jax.ShapeDtypeStruct((N, F), jnp.float32),
            jax.ShapeDtypeStruct((N, F), jnp.float8_e4m3fn),
            jax.ShapeDtypeStruct((N, N), jnp.float8_e4m3fn),
        ],
    )(weight_b, A, A, s1, b1_2d, w2_bf)

    out = pl.pallas_call(
        _pass2_body,
        grid=(N // TM,),
        in_specs=[
            pl.BlockSpec(memory_space=pltpu.SMEM),
            pl.BlockSpec((TM, N), lambda i: (i, 0)),
            pl.BlockSpec((N, F), lambda i: (0, 0)),
            pl.BlockSpec((TM, F), lambda i: (i, 0)),
            pl.BlockSpec((1, F), lambda i: (0, 0)),
        ],
        out_specs=pl.BlockSpec((TM, F), lambda i: (i, 0)),
        out_shape=jax.ShapeDtypeStruct((N, F), jnp.float32),
    )(weight_a, fa8, s2, u1, b2_2d)

    return out


# speedup vs baseline: 1.4359x; 1.0236x over previous
import jax
import jax.numpy as jnp
from jax.experimental import pallas as pl
from jax.experimental.pallas import tpu as pltpu

N = 10000
F = 128
TM = 200


def _pass1_body(wb_ref, a0_ref, a1_ref, f_ref, w1_ref, b1_ref, w2_ref,
                u1_ref, s2_ref, fa8_ref, s1_ref):
    @pl.when(pl.program_id(0) == 0)
    def _():
        fb = f_ref[...].astype(jnp.bfloat16)
        s1_ref[...] = jnp.dot(fb, w1_ref[...],
                              preferred_element_type=jnp.float32
                              ).astype(jnp.bfloat16)

    wb0 = wb_ref[0, 0]
    wb1 = wb_ref[1, 0]
    m = a0_ref[0] * wb0 + a1_ref[0] * wb1
    fa8_ref[...] = m.astype(jnp.float8_e4m3fn)
    mb = m.astype(jnp.bfloat16)
    u1 = jnp.dot(mb, s1_ref[...], preferred_element_type=jnp.float32)
    u1 = u1 + b1_ref[...]
    u1_ref[...] = u1
    s2_ref[...] = jnp.dot(u1.astype(jnp.bfloat16), w2_ref[...],
                          preferred_element_type=jnp.float32).astype(
                              jnp.float8_e4m3fn)


def _pass2_body(wa_ref, fa8_ref, s2_ref, u1_ref, b2_ref, o_ref):
    wa = wa_ref[0, 0]
    u2 = jnp.dot(fa8_ref[...], s2_ref[...],
                 preferred_element_type=jnp.float32)
    u2 = u2 + b2_ref[...]
    o_ref[...] = (u1_ref[...] + u2 * wa) * 0.5


def kernel(feature, A, W1, b1, W2, b2, weight_b, weight_a):
    w1_bf = W1.astype(jnp.bfloat16)
    w2_bf = W2.astype(jnp.bfloat16)
    b1_2d = b1.reshape(1, F)
    b2_2d = b2.reshape(1, F)

    u1, s2, fa8 = pl.pallas_call(
        _pass1_body,
        grid=(N // TM,),
        in_specs=[
            pl.BlockSpec(memory_space=pltpu.SMEM),
            pl.BlockSpec((1, TM, N), lambda i: (0, i, 0)),
            pl.BlockSpec((1, TM, N), lambda i: (1, i, 0)),
            pl.BlockSpec((N, F), lambda i: (0, 0)),
            pl.BlockSpec((F, F), lambda i: (0, 0)),
            pl.BlockSpec((1, F), lambda i: (0, 0)),
            pl.BlockSpec((F, F), lambda i: (0, 0)),
        ],
        out_specs=[
            pl.BlockSpec((TM, F), lambda i: (i, 0)),
            pl.BlockSpec((TM, F), lambda i: (i, 0)),
            pl.BlockSpec((TM, N), lambda i: (i, 0)),
        ],
        out_shape=[
            jax.ShapeDtypeStruct((N, F), jnp.float32),
            jax.ShapeDtypeStruct((N, F), jnp.float8_e4m3fn),
            jax.ShapeDtypeStruct((N, N), jnp.float8_e4m3fn),
        ],
        scratch_shapes=[pltpu.VMEM((N, F), jnp.bfloat16)],
    )(weight_b, A, A, feature, w1_bf, b1_2d, w2_bf)

    out = pl.pallas_call(
        _pass2_body,
        grid=(N // TM,),
        in_specs=[
            pl.BlockSpec(memory_space=pltpu.SMEM),
            pl.BlockSpec((TM, N), lambda i: (i, 0)),
            pl.BlockSpec((N, F), lambda i: (0, 0)),
            pl.BlockSpec((TM, F), lambda i: (i, 0)),
            pl.BlockSpec((1, F), lambda i: (0, 0)),
        ],
        out_specs=pl.BlockSpec((TM, F), lambda i: (i, 0)),
        out_shape=jax.ShapeDtypeStruct((N, F), jnp.float32),
    )(weight_a, fa8, s2, u1, b2_2d)

    return out


# speedup vs baseline: 1.4363x; 1.0002x over previous
import jax
import jax.numpy as jnp
from jax.experimental import pallas as pl
from jax.experimental.pallas import tpu as pltpu

N = 10000
F = 128
TM = 200
NB = N // TM


def _body(wb_ref, wa_ref, a0_ref, a1_ref, f_ref, w1_ref, b1_ref, w2_ref,
          b2_ref, fa8_ref, o_ref,
          s1_ref, u1_ref, s2_ref, spill_ref, fetch_ref, sem_out, sem_in):
    i = pl.program_id(0)

    @pl.when(i == 0)
    def _():
        s1_ref[...] = jnp.dot(f_ref[...], w1_ref[...],
                              preferred_element_type=jnp.float32
                              ).astype(jnp.bfloat16)

    @pl.when(i < NB)
    def _phase1():
        slot = jax.lax.rem(i, 2)

        @pl.when(i >= 2)
        def _():
            pltpu.make_async_copy(
                spill_ref.at[slot],
                fa8_ref.at[pl.ds((i - 2) * TM, TM), :],
                sem_out.at[slot]).wait()

        wb0 = wb_ref[0, 0]
        wb1 = wb_ref[1, 0]
        m = a0_ref[0] * wb0 + a1_ref[0] * wb1
        spill_ref[slot] = m.astype(jnp.float8_e4m3fn)
        pltpu.make_async_copy(
            spill_ref.at[slot],
            fa8_ref.at[pl.ds(i * TM, TM), :],
            sem_out.at[slot]).start()
        mb = m.astype(jnp.bfloat16)
        u1 = jnp.dot(mb, s1_ref[...], preferred_element_type=jnp.float32)
        u1 = u1 + b1_ref[...]
        u1_ref[pl.ds(i * TM, TM), :] = u1
        s2_ref[pl.ds(i * TM, TM), :] = jnp.dot(
            u1.astype(jnp.bfloat16), w2_ref[...],
            preferred_element_type=jnp.float32).astype(jnp.float8_e4m3fn)

    @pl.when(i >= NB)
    def _phase2():
        j = i - NB
        slot = jax.lax.rem(j, 2)

        @pl.when(j < 2)
        def _():
            pltpu.make_async_copy(
                spill_ref.at[slot],
                fa8_ref.at[pl.ds((NB - 2 + j) * TM, TM), :],
                sem_out.at[slot]).wait()

        @pl.when(j == 0)
        def _():
            pltpu.make_async_copy(
                fa8_ref.at[pl.ds(0, TM), :],
                fetch_ref.at[0],
                sem_in.at[0]).start()

        @pl.when(j < NB - 1)
        def _():
            nxt = jax.lax.rem(j + 1, 2)
            pltpu.make_async_copy(
                fa8_ref.at[pl.ds((j + 1) * TM, TM), :],
                fetch_ref.at[nxt],
                sem_in.at[nxt]).start()

        pltpu.make_async_copy(
            fa8_ref.at[pl.ds(j * TM, TM), :],
            fetch_ref.at[slot],
            sem_in.at[slot]).wait()

        wa = wa_ref[0, 0]
        u2 = jnp.dot(fetch_ref[slot], s2_ref[...],
                     preferred_element_type=jnp.float32)
        u2 = u2 + b2_ref[...]
        o_ref[...] = (u1_ref[pl.ds(j * TM, TM), :] + u2 * wa) * 0.5


def kernel(feature, A, W1, b1, W2, b2, weight_b, weight_a):
    f_bf = feature.astype(jnp.bfloat16)
    w1_bf = W1.astype(jnp.bfloat16)
    w2_bf = W2.astype(jnp.bfloat16)
    b1_2d = b1.reshape(1, F)
    b2_2d = b2.reshape(1, F)

    _, out = pl.pallas_call(
        _body,
        grid=(2 * NB,),
        in_specs=[
            pl.BlockSpec(memory_space=pltpu.SMEM),
            pl.BlockSpec(memory_space=pltpu.SMEM),
            pl.BlockSpec((1, TM, N), lambda i: (0, jnp.minimum(i, NB - 1), 0)),
            pl.BlockSpec((1, TM, N), lambda i: (1, jnp.minimum(i, NB - 1), 0)),
            pl.BlockSpec((N, F), lambda i: (0, 0)),
            pl.BlockSpec((F, F), lambda i: (0, 0)),
            pl.BlockSpec((1, F), lambda i: (0, 0)),
            pl.BlockSpec((F, F), lambda i: (0, 0)),
            pl.BlockSpec((1, F), lambda i: (0, 0)),
        ],
        out_specs=[
            pl.BlockSpec(memory_space=pltpu.MemorySpace.HBM),
            pl.BlockSpec((TM, F), lambda i: (jnp.maximum(i - NB, 0), 0)),
        ],
        out_shape=[
            jax.ShapeDtypeStruct((N, N), jnp.float8_e4m3fn),
            jax.ShapeDtypeStruct((N, F), jnp.float32),
        ],
        scratch_shapes=[
            pltpu.VMEM((N, F), jnp.bfloat16),
            pltpu.VMEM((N, F), jnp.float32),
            pltpu.VMEM((N, F), jnp.float8_e4m3fn),
            pltpu.VMEM((2, TM, N), jnp.float8_e4m3fn),
            pltpu.VMEM((2, TM, N), jnp.float8_e4m3fn),
            pltpu.SemaphoreType.DMA((2,)),
            pltpu.SemaphoreType.DMA((2,)),
        ],
    )(weight_b, weight_a, A, A, f_bf, w1_bf, b1_2d, w2_bf, b2_2d)

    return out
